# baseline (speedup 1.0000x reference)
import functools

import jax
import jax.numpy as jnp
from jax import lax
from jax.experimental import pallas as pl
from jax.experimental.pallas import tpu as pltpu
from jax.experimental.pallas import tpu_sc as plsc

_V = 1000000
_H = 128
_B = 1024
_T = 200
_L = 2

_NW = 32
_CH = 128

_S = 8
_CHUNKS = (16, 24, 40, 56, 64)


def _sc_gather(table, idx3, nchc):
    mesh = plsc.VectorSubcoreMesh(core_axis_name="c", subcore_axis_name="s")
    n_rows = _NW * nchc * _CH

    @functools.partial(
        pl.kernel,
        mesh=mesh,
        out_type=jax.ShapeDtypeStruct((n_rows, _H), jnp.float32),
        scratch_types=[
            pltpu.VMEM((nchc, _CH), jnp.int32),
            pltpu.VMEM((2, _CH, _H), jnp.float32),
            pltpu.SemaphoreType.DMA,
        ],
    )
    def gather_k(table_hbm, idx_hbm, out_hbm, idx_v, buf_v, sem):
        wid = lax.axis_index("s") * 2 + lax.axis_index("c")
        pltpu.sync_copy(idx_hbm.at[wid], idx_v)
        base = wid * (nchc * _CH)

        pltpu.async_copy(table_hbm.at[idx_v.at[0]], buf_v.at[0], sem)

        def body(j, carry):
            cur = buf_v.at[j % 2]
            pltpu.make_async_copy(table_hbm.at[idx_v.at[j]], cur, sem).wait()

            @pl.when(j + 1 < nchc)
            def _prefetch():
                pltpu.async_copy(table_hbm.at[idx_v.at[j + 1]],
                                 buf_v.at[(j + 1) % 2], sem)

            pltpu.sync_copy(cur, out_hbm.at[pl.ds(base + j * _CH, _CH)])
            return carry

        lax.fori_loop(0, nchc, body, 0)

    return gather_k(table, idx3)


def _lstm_body(tbc, x_ref, h_ref, c_ref, w0_ref, b0_ref, w1_ref, b1_ref,
               out_ref, hn_ref, cn_ref,
               h0s, c0s, h1s, c1s):
    tb = pl.program_id(0)

    @pl.when(tb == 0)
    def _init():
        h0s[...] = h_ref[0]
        c0s[...] = c_ref[0]
        h1s[...] = h_ref[1]
        c1s[...] = c_ref[1]

    def cell(xt, hs, cs, w, b):
        z = jnp.concatenate([xt, hs], axis=1)
        g = jnp.dot(z, w, preferred_element_type=jnp.float32) + b
        ti = jnp.tanh(g[:, 0 * _H:1 * _H])
        tf = jnp.tanh(g[:, 1 * _H:2 * _H])
        gg = jnp.tanh(g[:, 2 * _H:3 * _H])
        to = jnp.tanh(g[:, 3 * _H:4 * _H])
        c = 0.5 * ((tf + 1.0) * cs + (ti + 1.0) * gg)
        h = 0.5 * jnp.tanh(c) * (to + 1.0)
        return h, c

    h0v, c0v = h0s[...], c0s[...]
    h1v, c1v = h1s[...], c1s[...]
    for i in range(_S):
        h0v, c0v = cell(x_ref[i], h0v, c0v, w0_ref[...], b0_ref[...])
        h1v, c1v = cell(h0v, h1v, c1v, w1_ref[...], b1_ref[...])
        out_ref[:, i, :] = h1v
    h0s[...] = h0v
    c0s[...] = c0v
    h1s[...] = h1v
    c1s[...] = c1v

    @pl.when(tb == tbc - 1)
    def _fin():
        hn_ref[0] = h0v
        hn_ref[1] = h1v
        cn_ref[0] = c0v
        cn_ref[1] = c1v


def _lstm_chunk(off, tbc, x, h_in, c_in, w0, b0, w1, b1, out_prev):
    full = lambda shape: pl.BlockSpec(shape, lambda t: (0,) * len(shape))
    in_specs = [
        pl.BlockSpec((_S, _B, _H), lambda t: (t, 0, 0)),
        full((_L, _B, _H)),
        full((_L, _B, _H)),
        full((2 * _H, 4 * _H)),
        full((1, 4 * _H)),
        full((2 * _H, 4 * _H)),
        full((1, 4 * _H)),
    ]
    args = [x, h_in, c_in, w0, b0, w1, b1]
    aliases = {}
    body = functools.partial(_lstm_body, tbc)
    if out_prev is not None:
        in_specs.append(pl.BlockSpec(memory_space=pl.ANY))
        args.append(out_prev)
        aliases = {7: 0}

        def body(x_ref, h_ref, c_ref, w0_ref, b0_ref, w1_ref, b1_ref,
                 out_prev_ref, out_ref, hn_ref, cn_ref, *scratch):
            del out_prev_ref
            _lstm_body(tbc, x_ref, h_ref, c_ref, w0_ref, b0_ref, w1_ref,
                       b1_ref, out_ref, hn_ref, cn_ref, *scratch)

    return pl.pallas_call(
        body,
        grid=(tbc,),
        in_specs=in_specs,
        out_specs=[
            pl.BlockSpec((_B, _S, _H), lambda t: (0, off + t, 0)),
            full((_L, _B, _H)),
            full((_L, _B, _H)),
        ],
        out_shape=[
            jax.ShapeDtypeStruct((_B, _T, _H), jnp.float32),
            jax.ShapeDtypeStruct((_L, _B, _H), jnp.float32),
            jax.ShapeDtypeStruct((_L, _B, _H), jnp.float32),
        ],
        scratch_shapes=[pltpu.VMEM((_B, _H), jnp.float32)] * 4,
        input_output_aliases=aliases,
        compiler_params=pltpu.CompilerParams(
            dimension_semantics=("arbitrary",)),
    )(*args)


def kernel(batch_input, h0, c0, table,
           Wih0, Whh0, bih0, bhh0, Wih1, Whh1, bih1, bhh1):
    idx = jnp.transpose(batch_input).astype(jnp.int32)
    gate_scale = jnp.concatenate([
        jnp.full((1, 2 * _H), 0.5, jnp.float32),
        jnp.ones((1, _H), jnp.float32),
        jnp.full((1, _H), 0.5, jnp.float32),
    ], axis=1)
    b0 = (bih0 + bhh0).reshape(1, 4 * _H) * gate_scale
    b1 = (bih1 + bhh1).reshape(1, 4 * _H) * gate_scale
    w0 = jnp.concatenate([Wih0.T, Whh0.T], axis=0) * gate_scale
    w1 = jnp.concatenate([Wih1.T, Whh1.T], axis=0) * gate_scale

    xs = []
    t0 = 0
    for steps in _CHUNKS:
        nchc = (steps * _B) // (_NW * _CH)
        idx3 = idx[t0:t0 + steps].reshape(_NW, nchc, _CH)
        xs.append(_sc_gather(table, idx3, nchc).reshape(steps, _B, _H))
        t0 += steps

    out, hn, cn = None, h0, c0
    t0 = 0
    for x, steps in zip(xs, _CHUNKS):
        out, hn, cn = _lstm_chunk(t0 // _S, steps // _S, x, hn, cn,
                                  w0, b0, w1, b1, out)
        t0 += steps
    return out, hn, cn


# speedup vs baseline: 1.0911x; 1.0911x over previous
import functools

import jax
import jax.numpy as jnp
from jax import lax
from jax.experimental import pallas as pl
from jax.experimental.pallas import tpu as pltpu
from jax.experimental.pallas import tpu_sc as plsc

_V = 1000000
_H = 128
_B = 1024
_T = 200
_L = 2

_NW = 32
_CH = 128

_S = 8
_CHUNKS = (16, 32, 64, 88)


def _sc_gather(table, idx3, nchc):
    mesh = plsc.VectorSubcoreMesh(core_axis_name="c", subcore_axis_name="s")
    n_rows = _NW * nchc * _CH

    @functools.partial(
        pl.kernel,
        mesh=mesh,
        out_type=jax.ShapeDtypeStruct((n_rows, _H), jnp.float32),
        scratch_types=[
            pltpu.VMEM((nchc, _CH), jnp.int32),
            pltpu.VMEM((2, _CH, _H), jnp.float32),
            pltpu.SemaphoreType.DMA,
        ],
    )
    def gather_k(table_hbm, idx_hbm, out_hbm, idx_v, buf_v, sem):
        wid = lax.axis_index("s") * 2 + lax.axis_index("c")
        pltpu.sync_copy(idx_hbm.at[wid], idx_v)
        base = wid * (nchc * _CH)

        pltpu.async_copy(table_hbm.at[idx_v.at[0]], buf_v.at[0], sem)

        def body(j, carry):
            cur = buf_v.at[j % 2]
            pltpu.make_async_copy(table_hbm.at[idx_v.at[j]], cur, sem).wait()

            @pl.when(j + 1 < nchc)
            def _prefetch():
                pltpu.async_copy(table_hbm.at[idx_v.at[j + 1]],
                                 buf_v.at[(j + 1) % 2], sem)

            pltpu.sync_copy(cur, out_hbm.at[pl.ds(base + j * _CH, _CH)])
            return carry

        lax.fori_loop(0, nchc, body, 0)

    return gather_k(table, idx3)


def _lstm_body(tbc, x_ref, h_ref, c_ref, w0_ref, b0_ref, w1_ref, b1_ref,
               out_ref, hn_ref, cn_ref,
               h0s, c0s, h1s, c1s):
    tb = pl.program_id(0)

    @pl.when(tb == 0)
    def _init():
        h0s[...] = h_ref[0]
        c0s[...] = c_ref[0]
        h1s[...] = h_ref[1]
        c1s[...] = c_ref[1]

    def cell(xt, hs, cs, w, b):
        z = jnp.concatenate([xt, hs], axis=1)
        g = jnp.dot(z, w, preferred_element_type=jnp.float32) + b
        ti = jnp.tanh(g[:, 0 * _H:1 * _H])
        tf = jnp.tanh(g[:, 1 * _H:2 * _H])
        gg = jnp.tanh(g[:, 2 * _H:3 * _H])
        to = jnp.tanh(g[:, 3 * _H:4 * _H])
        c = 0.5 * ((tf + 1.0) * cs + (ti + 1.0) * gg)
        h = 0.5 * jnp.tanh(c) * (to + 1.0)
        return h, c

    h0v, c0v = h0s[...], c0s[...]
    h1v, c1v = h1s[...], c1s[...]
    for i in range(_S):
        h0v, c0v = cell(x_ref[i], h0v, c0v, w0_ref[...], b0_ref[...])
        h1v, c1v = cell(h0v, h1v, c1v, w1_ref[...], b1_ref[...])
        out_ref[:, i, :] = h1v
    h0s[...] = h0v
    c0s[...] = c0v
    h1s[...] = h1v
    c1s[...] = c1v

    @pl.when(tb == tbc - 1)
    def _fin():
        hn_ref[0] = h0v
        hn_ref[1] = h1v
        cn_ref[0] = c0v
        cn_ref[1] = c1v


def _lstm_chunk(off, tbc, x, h_in, c_in, w0, b0, w1, b1, out_prev):
    full = lambda shape: pl.BlockSpec(shape, lambda t: (0,) * len(shape))
    in_specs = [
        pl.BlockSpec((_S, _B, _H), lambda t: (t, 0, 0)),
        full((_L, _B, _H)),
        full((_L, _B, _H)),
        full((2 * _H, 4 * _H)),
        full((1, 4 * _H)),
        full((2 * _H, 4 * _H)),
        full((1, 4 * _H)),
    ]
    args = [x, h_in, c_in, w0, b0, w1, b1]
    aliases = {}
    body = functools.partial(_lstm_body, tbc)
    if out_prev is not None:
        in_specs.append(pl.BlockSpec(memory_space=pl.ANY))
        args.append(out_prev)
        aliases = {7: 0}

        def body(x_ref, h_ref, c_ref, w0_ref, b0_ref, w1_ref, b1_ref,
                 out_prev_ref, out_ref, hn_ref, cn_ref, *scratch):
            del out_prev_ref
            _lstm_body(tbc, x_ref, h_ref, c_ref, w0_ref, b0_ref, w1_ref,
                       b1_ref, out_ref, hn_ref, cn_ref, *scratch)

    return pl.pallas_call(
        body,
        grid=(tbc,),
        in_specs=in_specs,
        out_specs=[
            pl.BlockSpec((_B, _S, _H), lambda t: (0, off + t, 0)),
            full((_L, _B, _H)),
            full((_L, _B, _H)),
        ],
        out_shape=[
            jax.ShapeDtypeStruct((_B, _T, _H), jnp.float32),
            jax.ShapeDtypeStruct((_L, _B, _H), jnp.float32),
            jax.ShapeDtypeStruct((_L, _B, _H), jnp.float32),
        ],
        scratch_shapes=[pltpu.VMEM((_B, _H), jnp.float32)] * 4,
        input_output_aliases=aliases,
        compiler_params=pltpu.CompilerParams(
            dimension_semantics=("arbitrary",)),
    )(*args)


def kernel(batch_input, h0, c0, table,
           Wih0, Whh0, bih0, bhh0, Wih1, Whh1, bih1, bhh1):
    idx = jnp.transpose(batch_input).astype(jnp.int32)
    gate_scale = jnp.concatenate([
        jnp.full((1, 2 * _H), 0.5, jnp.float32),
        jnp.ones((1, _H), jnp.float32),
        jnp.full((1, _H), 0.5, jnp.float32),
    ], axis=1)
    b0 = (bih0 + bhh0).reshape(1, 4 * _H) * gate_scale
    b1 = (bih1 + bhh1).reshape(1, 4 * _H) * gate_scale
    w0 = jnp.concatenate([Wih0.T, Whh0.T], axis=0) * gate_scale
    w1 = jnp.concatenate([Wih1.T, Whh1.T], axis=0) * gate_scale

    xs = []
    t0 = 0
    for steps in _CHUNKS:
        nchc = (steps * _B) // (_NW * _CH)
        idx3 = idx[t0:t0 + steps].reshape(_NW, nchc, _CH)
        xs.append(_sc_gather(table, idx3, nchc).reshape(steps, _B, _H))
        t0 += steps

    out, hn, cn = None, h0, c0
    t0 = 0
    for x, steps in zip(xs, _CHUNKS):
        out, hn, cn = _lstm_chunk(t0 // _S, steps // _S, x, hn, cn,
                                  w0, b0, w1, b1, out)
        t0 += steps
    return out, hn, cn
